# Initial kernel scaffold; baseline (speedup 1.0000x reference)
import functools

import jax
import jax.numpy as jnp
from jax import lax
from jax.experimental import pallas as pl
from jax.experimental.pallas import tpu as pltpu
from jax.experimental.pallas import tpu_sc as plsc

N = 10000
D = 128
E = 320000
K = 128

NC = 2
NS = 16
NW = NC * NS
EPW = E // NW
CH = 80
NCHUNK = EPW // CH
RPT = N // NS
ZCH = 125
NZ = RPT // ZCH

SROW = 80
SCOL = 125


def _sc_agg(nodes, src, dst, adj):
    mesh = plsc.VectorSubcoreMesh(core_axis_name="c", subcore_axis_name="s")

    @functools.partial(
        pl.kernel,
        out_type=jax.ShapeDtypeStruct((NC * N, D), jnp.float32),
        mesh=mesh,
        scratch_types=dict(
            agg_sh=pltpu.VMEM_SHARED((N, D), jnp.float32),
            ebuf=pltpu.VMEM((CH, D), jnp.float32),
            sidx=pltpu.VMEM((CH,), jnp.int32),
            didx=pltpu.VMEM((CH,), jnp.int32),
            adjv=pltpu.VMEM((CH,), jnp.float32),
            zbuf=pltpu.VMEM((ZCH, D), jnp.float32),
            sem=pltpu.SemaphoreType.DMA,
        ),
    )
    def body(nodes_hbm, src_hbm, dst_hbm, adj_hbm, out_hbm,
             agg_sh, ebuf, sidx, didx, adjv, zbuf, sem):
        c = lax.axis_index("c")
        s = lax.axis_index("s")
        wid = s * NC + c

        zeros16 = jnp.zeros((16,), jnp.float32)

        def zrow(r, carry):
            for q in range(D // 16):
                zbuf[r, pl.ds(q * 16, 16)] = zeros16
            return carry

        lax.fori_loop(0, ZCH, zrow, 0)
        for z in range(NZ):
            pltpu.sync_copy(zbuf, agg_sh.at[pl.ds(s * RPT + z * ZCH, ZCH)])
        plsc.subcore_barrier()

        def chunk(g, carry):
            base = wid * EPW + g * CH
            pltpu.sync_copy(src_hbm.at[pl.ds(base, CH)], sidx)
            pltpu.sync_copy(dst_hbm.at[pl.ds(base, CH)], didx)
            pltpu.sync_copy(adj_hbm.at[pl.ds(base, CH)], adjv)
            pltpu.async_copy(nodes_hbm.at[sidx], ebuf, sem).wait()
            for j in range(CH // 16):
                av = adjv[pl.ds(j * 16, 16)]
                for l in range(16):
                    a = jnp.take(av, jnp.full((16,), l, jnp.int32),
                                 mode="promise_in_bounds")
                    r = j * 16 + l
                    for q in range(D // 16):
                        ebuf[r, pl.ds(q * 16, 16)] = (
                            ebuf[r, pl.ds(q * 16, 16)] * a)
            pltpu.sync_copy(ebuf, agg_sh.at[didx], add=True)
            return carry

        lax.fori_loop(0, NCHUNK, chunk, 0)
        plsc.subcore_barrier()

        for z in range(NZ):
            r0 = s * RPT + z * ZCH
            pltpu.sync_copy(agg_sh.at[pl.ds(r0, ZCH)], zbuf)
            pltpu.sync_copy(zbuf, out_hbm.at[pl.ds(c * N + r0, ZCH)])

    return body(nodes, src, dst, adj)


def _scores_body(p_ref, nb_ref, o_ref):
    p = p_ref[...]
    pn = p / (jnp.sqrt(jnp.sum(p * p)) + 1e-8)
    o_ref[...] = lax.dot_general(
        pn, nb_ref[0],
        (((1,), (1,)), ((), ())),
        preferred_element_type=jnp.float32)


def _scores_call(p2, nodes3):
    return pl.pallas_call(
        _scores_body,
        grid=(SROW,),
        in_specs=[
            pl.BlockSpec((1, D), lambda a: (0, 0)),
            pl.BlockSpec((1, SCOL, D), lambda a: (a, 0, 0)),
        ],
        out_specs=pl.BlockSpec((1, SCOL), lambda a: (a, 0)),
        out_shape=jax.ShapeDtypeStruct((SROW, SCOL), jnp.float32),
    )(p2, nodes3)


def _gru_body(sc_ref, nodes_ref, w_ref,
              wz_ref, uz_ref, bz_ref, wr_ref, ur_ref, br_ref,
              wh_ref, uh_ref, bh_ref, out_ref, x_ref):
    lin = (lax.broadcasted_iota(jnp.int32, (SROW, SCOL), 0) * SCOL
           + lax.broadcasted_iota(jnp.int32, (SROW, SCOL), 1))

    def step(t, S):
        m = jnp.max(S)
        amin = jnp.min(jnp.where(S == m, lin, jnp.int32(2**30)))
        row = nodes_ref[pl.ds(amin, 1), :]
        x_ref[pl.ds(t, 1), :] = row * jnp.tanh(m)
        return jnp.where(lin == amin, -jnp.inf, S)

    lax.fori_loop(0, K, step, sc_ref[...])

    X = x_ref[...]
    H = w_ref[...]
    dgT = lambda A, B: lax.dot_general(
        A, B, (((1,), (1,)), ((), ())), preferred_element_type=jnp.float32)
    mm = lambda A, B: jnp.dot(A, B, preferred_element_type=jnp.float32)
    Z = jax.nn.sigmoid(dgT(wz_ref[...], X) + mm(uz_ref[...], H) + bz_ref[...])
    R = jax.nn.sigmoid(dgT(wr_ref[...], X) + mm(ur_ref[...], H) + br_ref[...])
    Ht = jnp.tanh(dgT(wh_ref[...], X) + mm(uh_ref[...], R * H) + bh_ref[...])
    out_ref[...] = (1.0 - Z) * H + Z * Ht


def _gru_call(scores, nodes, weights, Wz, Uz, bz, Wr, Ur, br, Wh, Uh, bh):
    return pl.pallas_call(
        _gru_body,
        out_shape=jax.ShapeDtypeStruct((D, K), jnp.float32),
        scratch_shapes=[pltpu.VMEM((K, D), jnp.float32)],
    )(scores, nodes, weights, Wz, Uz, bz, Wr, Ur, br, Wh, Uh, bh)


_MM_BLK = 1000


def _mm_body(a0_ref, a1_ref, w_ref, o_ref):
    acc = a0_ref[...] + a1_ref[...]
    o_ref[...] = jnp.maximum(
        jnp.dot(acc, w_ref[...], preferred_element_type=jnp.float32), 0.0)


def _mm_call(a0, a1, w):
    return pl.pallas_call(
        _mm_body,
        grid=(N // _MM_BLK,),
        in_specs=[
            pl.BlockSpec((_MM_BLK, D), lambda i: (i, 0)),
            pl.BlockSpec((_MM_BLK, D), lambda i: (i, 0)),
            pl.BlockSpec((D, K), lambda i: (0, 0)),
        ],
        out_specs=pl.BlockSpec((_MM_BLK, K), lambda i: (i, 0)),
        out_shape=jax.ShapeDtypeStruct((N, K), jnp.float32),
    )(a0, a1, w)


def kernel(nodes, adj_values, weights, p, Wz, Uz, bz, Wr, Ur, br, Wh, Uh, bh,
           edge_index):
    src = edge_index[0]
    dst = edge_index[1]
    agg2 = _sc_agg(nodes, src, dst, adj_values)
    scores = _scores_call(p.reshape(1, D), nodes.reshape(SROW, SCOL, D))
    weights_new = _gru_call(scores, nodes, weights,
                            Wz, Uz, bz, Wr, Ur, br, Wh, Uh, bh)
    nodes_new = _mm_call(agg2[:N], agg2[N:], weights_new)
    return (nodes_new, weights_new)


# speedup vs baseline: 4.1309x; 4.1309x over previous
import functools

import jax
import jax.numpy as jnp
from jax import lax
from jax.experimental import pallas as pl
from jax.experimental.pallas import tpu as pltpu
from jax.experimental.pallas import tpu_sc as plsc

N = 10000
D = 128
E = 320000
K = 128

NC = 2
NS = 16
NW = NC * NS
EPW = E // NW
CH = 80
NCHUNK = EPW // CH
OWN = 624
WCH = 208
NWCH = OWN // WCH
TAIL = N - NS * OWN

SROW = 80
SCOL = 125


def _sc_agg(nodes, src, dst, adj):
    mesh = plsc.VectorSubcoreMesh(core_axis_name="c", subcore_axis_name="s")

    @functools.partial(
        pl.kernel,
        out_type=jax.ShapeDtypeStruct((NC * N, D), jnp.float32),
        mesh=mesh,
        scratch_types=dict(
            agg_sh=pltpu.VMEM_SHARED((N, D), jnp.float32),
            ebuf=pltpu.VMEM((CH, D), jnp.float32),
            sidx=pltpu.VMEM((CH,), jnp.int32),
            didx=pltpu.VMEM((CH,), jnp.int32),
            adjv=pltpu.VMEM((CH,), jnp.float32),
            zbuf=pltpu.VMEM((WCH, D), jnp.float32),
            sem=pltpu.SemaphoreType.DMA,
        ),
    )
    def body(nodes_hbm, src_hbm, dst_hbm, adj_hbm, out_hbm,
             agg_sh, ebuf, sidx, didx, adjv, zbuf, sem):
        c = lax.axis_index("c")
        s = lax.axis_index("s")
        wid = s * NC + c

        zeros16 = jnp.zeros((16,), jnp.float32)

        def zrow(r, carry):
            for q in range(D // 16):
                zbuf[r, pl.ds(q * 16, 16)] = zeros16
            return carry

        lax.fori_loop(0, WCH, zrow, 0)
        for z in range(NWCH):
            pltpu.sync_copy(zbuf, agg_sh.at[pl.ds(s * OWN + z * WCH, WCH)])

        @pl.when(s == NS - 1)
        def _zero_tail():
            pltpu.sync_copy(zbuf.at[pl.ds(0, TAIL)],
                            agg_sh.at[pl.ds(NS * OWN, TAIL)])

        plsc.subcore_barrier()

        def chunk(g, carry):
            base = wid * EPW + g * CH
            pltpu.sync_copy(src_hbm.at[pl.ds(base, CH)], sidx)
            pltpu.sync_copy(dst_hbm.at[pl.ds(base, CH)], didx)
            pltpu.sync_copy(adj_hbm.at[pl.ds(base, CH)], adjv)
            pltpu.async_copy(nodes_hbm.at[sidx], ebuf, sem).wait()
            for j in range(CH // 16):
                av = adjv[pl.ds(j * 16, 16)]
                for l in range(16):
                    a = av.at[jnp.full((16,), l, jnp.int32)].get(
                        mode="promise_in_bounds")
                    r = j * 16 + l
                    for q in range(D // 16):
                        ebuf[r, pl.ds(q * 16, 16)] = (
                            ebuf[r, pl.ds(q * 16, 16)] * a)
            pltpu.sync_copy(ebuf, agg_sh.at[didx], add=True)
            return carry

        lax.fori_loop(0, NCHUNK, chunk, 0)
        plsc.subcore_barrier()

        for z in range(NWCH):
            r0 = s * OWN + z * WCH
            pltpu.sync_copy(agg_sh.at[pl.ds(r0, WCH)], zbuf)
            pltpu.sync_copy(zbuf, out_hbm.at[pl.ds(c * N + r0, WCH)])

        @pl.when(s == NS - 1)
        def _write_tail():
            pltpu.sync_copy(agg_sh.at[pl.ds(NS * OWN, TAIL)],
                            zbuf.at[pl.ds(0, TAIL)])
            pltpu.sync_copy(zbuf.at[pl.ds(0, TAIL)],
                            out_hbm.at[pl.ds(c * N + NS * OWN, TAIL)])

    return body(nodes, src, dst, adj)


def _scores_body(p_ref, n3_ref, o_ref):
    p = p_ref[...]
    pn = p / (jnp.sqrt(jnp.sum(p * p)) + 1e-8)
    for a in range(SROW):
        o_ref[pl.ds(a, 1), :] = lax.dot_general(
            pn, n3_ref[a],
            (((1,), (1,)), ((), ())),
            preferred_element_type=jnp.float32)


def _scores_call(p2, nodes3):
    return pl.pallas_call(
        _scores_body,
        out_shape=jax.ShapeDtypeStruct((SROW, SCOL), jnp.float32),
    )(p2, nodes3)


def _gru_body(sc_ref, nodes_ref, w_ref,
              wz_ref, uz_ref, bz_ref, wr_ref, ur_ref, br_ref,
              wh_ref, uh_ref, bh_ref, out_ref, x_ref):
    lin = (lax.broadcasted_iota(jnp.int32, (SROW, SCOL), 0) * SCOL
           + lax.broadcasted_iota(jnp.int32, (SROW, SCOL), 1))

    def step(t, S):
        m = jnp.max(S)
        amin = jnp.min(jnp.where(S == m, lin, jnp.int32(2**30)))
        row = nodes_ref[pl.ds(amin, 1), :]
        x_ref[pl.ds(t, 1), :] = row * jnp.tanh(m)
        return jnp.where(lin == amin, -jnp.inf, S)

    lax.fori_loop(0, K, step, sc_ref[...])

    X = x_ref[...]
    H = w_ref[...]
    dgT = lambda A, B: lax.dot_general(
        A, B, (((1,), (1,)), ((), ())), preferred_element_type=jnp.float32)
    mm = lambda A, B: jnp.dot(A, B, preferred_element_type=jnp.float32)
    Z = jax.nn.sigmoid(dgT(wz_ref[...], X) + mm(uz_ref[...], H) + bz_ref[...])
    R = jax.nn.sigmoid(dgT(wr_ref[...], X) + mm(ur_ref[...], H) + br_ref[...])
    Ht = jnp.tanh(dgT(wh_ref[...], X) + mm(uh_ref[...], R * H) + bh_ref[...])
    out_ref[...] = (1.0 - Z) * H + Z * Ht


def _gru_call(scores, nodes, weights, Wz, Uz, bz, Wr, Ur, br, Wh, Uh, bh):
    return pl.pallas_call(
        _gru_body,
        out_shape=jax.ShapeDtypeStruct((D, K), jnp.float32),
        scratch_shapes=[pltpu.VMEM((K, D), jnp.float32)],
    )(scores, nodes, weights, Wz, Uz, bz, Wr, Ur, br, Wh, Uh, bh)


_MM_BLK = 1000


def _mm_body(a0_ref, a1_ref, w_ref, o_ref):
    acc = a0_ref[...] + a1_ref[...]
    o_ref[...] = jnp.maximum(
        jnp.dot(acc, w_ref[...], preferred_element_type=jnp.float32), 0.0)


def _mm_call(a0, a1, w):
    return pl.pallas_call(
        _mm_body,
        grid=(N // _MM_BLK,),
        in_specs=[
            pl.BlockSpec((_MM_BLK, D), lambda i: (i, 0)),
            pl.BlockSpec((_MM_BLK, D), lambda i: (i, 0)),
            pl.BlockSpec((D, K), lambda i: (0, 0)),
        ],
        out_specs=pl.BlockSpec((_MM_BLK, K), lambda i: (i, 0)),
        out_shape=jax.ShapeDtypeStruct((N, K), jnp.float32),
    )(a0, a1, w)


def kernel(nodes, adj_values, weights, p, Wz, Uz, bz, Wr, Ur, br, Wh, Uh, bh,
           edge_index):
    src = edge_index[0]
    dst = edge_index[1]
    agg2 = _sc_agg(nodes, src, dst, adj_values)
    scores = _scores_call(p.reshape(1, D), nodes.reshape(SROW, SCOL, D))
    weights_new = _gru_call(scores, nodes, weights,
                            Wz, Uz, bz, Wr, Ur, br, Wh, Uh, bh)
    nodes_new = _mm_call(agg2[:N], agg2[N:], weights_new)
    return (nodes_new, weights_new)


# speedup vs baseline: 8.8133x; 2.1335x over previous
import functools

import jax
import jax.numpy as jnp
from jax import lax
from jax.experimental import pallas as pl
from jax.experimental.pallas import tpu as pltpu
from jax.experimental.pallas import tpu_sc as plsc

N = 10000
D = 128
E = 320000
K = 128

NC = 2
NS = 16
NW = NC * NS
EPW = E // NW
CH = 80
PH = 63
NCHUNK = 2 * PH
EPAD = NW * NCHUNK * CH - E
OWN = 624
TAIL = N - NS * OWN

SROW = 80
SCOL = 125


def _sc_agg(nodes, src3, dst3, adj3):
    mesh = plsc.VectorSubcoreMesh(core_axis_name="c", subcore_axis_name="s")

    @functools.partial(
        pl.kernel,
        out_type=jax.ShapeDtypeStruct((NC * N, D), jnp.float32),
        mesh=mesh,
        scratch_types=dict(
            agg_sh=pltpu.VMEM_SHARED((N, D), jnp.float32),
            ebufa=pltpu.VMEM((CH, D), jnp.float32),
            ebufb=pltpu.VMEM((CH, D), jnp.float32),
            sidx2=pltpu.VMEM((PH, CH), jnp.int32),
            didx2=pltpu.VMEM((PH, CH), jnp.int32),
            adj2=pltpu.VMEM((PH, CH), jnp.float32),
            sema=pltpu.SemaphoreType.DMA,
            semb=pltpu.SemaphoreType.DMA,
        ),
    )
    def body(nodes_hbm, src_hbm, dst_hbm, adj_hbm, out_hbm,
             agg_sh, ebufa, ebufb, sidx2, didx2, adj2, sema, semb):
        c = lax.axis_index("c")
        s = lax.axis_index("s")
        wid = s * NC + c

        zeros16 = jnp.zeros((16,), jnp.float32)

        def zrow(r, carry):
            for q in range(D // 16):
                ebufa[r, pl.ds(q * 16, 16)] = zeros16
            return carry

        lax.fori_loop(0, CH, zrow, 0)
        for z in range(OWN // CH):
            pltpu.sync_copy(ebufa, agg_sh.at[pl.ds(s * OWN + z * CH, CH)])
        pltpu.sync_copy(ebufa.at[pl.ds(0, OWN % CH)],
                        agg_sh.at[pl.ds(s * OWN + (OWN // CH) * CH, OWN % CH)])

        @pl.when(s == NS - 1)
        def _zero_tail():
            pltpu.sync_copy(ebufa.at[pl.ds(0, TAIL)],
                            agg_sh.at[pl.ds(NS * OWN, TAIL)])

        plsc.subcore_barrier()

        def gather_start(j, ebuf, sem):
            pltpu.make_async_copy(nodes_hbm.at[sidx2.at[j]], ebuf, sem).start()

        def gather_wait(j, ebuf, sem):
            pltpu.make_async_copy(nodes_hbm.at[sidx2.at[j]], ebuf, sem).wait()

        def scale_scatter(j, ebuf):
            for jj in range(CH // 16):
                av = adj2[j, pl.ds(jj * 16, 16)]
                for l in range(16):
                    a = av.at[jnp.full((16,), l, jnp.int32)].get(
                        mode="promise_in_bounds")
                    r = jj * 16 + l
                    for q in range(D // 16):
                        ebuf[r, pl.ds(q * 16, 16)] = (
                            ebuf[r, pl.ds(q * 16, 16)] * a)
            pltpu.sync_copy(ebuf, agg_sh.at[didx2.at[j]], add=True)

        def run_phase(ph):
            nch = PH
            pltpu.sync_copy(src_hbm.at[wid, ph], sidx2)
            pltpu.sync_copy(dst_hbm.at[wid, ph], didx2)
            pltpu.sync_copy(adj_hbm.at[wid, ph], adj2)
            gather_start(0, ebufa, sema)
            gather_start(1, ebufb, semb)

            def pair(i, carry):
                g = 2 * i
                gather_wait(g, ebufa, sema)
                scale_scatter(g, ebufa)

                @pl.when(g + 2 < nch)
                def _prefetch_a():
                    gather_start(g + 2, ebufa, sema)

                gather_wait(g + 1, ebufb, semb)
                scale_scatter(g + 1, ebufb)

                @pl.when(g + 3 < nch)
                def _prefetch_b():
                    gather_start(g + 3, ebufb, semb)

                return carry

            lax.fori_loop(0, nch // 2, pair, 0)
            if nch % 2:
                gather_wait(nch - 1, ebufa, sema)
                scale_scatter(nch - 1, ebufa)

        run_phase(0)
        run_phase(1)
        plsc.subcore_barrier()

        for z in range(OWN // CH):
            r0 = s * OWN + z * CH
            pltpu.sync_copy(agg_sh.at[pl.ds(r0, CH)], ebufa)
            pltpu.sync_copy(ebufa, out_hbm.at[pl.ds(c * N + r0, CH)])
        r0 = s * OWN + (OWN // CH) * CH
        pltpu.sync_copy(agg_sh.at[pl.ds(r0, OWN % CH)],
                        ebufa.at[pl.ds(0, OWN % CH)])
        pltpu.sync_copy(ebufa.at[pl.ds(0, OWN % CH)],
                        out_hbm.at[pl.ds(c * N + r0, OWN % CH)])

        @pl.when(s == NS - 1)
        def _write_tail():
            pltpu.sync_copy(agg_sh.at[pl.ds(NS * OWN, TAIL)],
                            ebufb.at[pl.ds(0, TAIL)])
            pltpu.sync_copy(ebufb.at[pl.ds(0, TAIL)],
                            out_hbm.at[pl.ds(c * N + NS * OWN, TAIL)])

    return body(nodes, src3, dst3, adj3)


def _scores_body(p_ref, n3_ref, o_ref):
    p = p_ref[...]
    pn = p / (jnp.sqrt(jnp.sum(p * p)) + 1e-8)
    for a in range(SROW):
        o_ref[pl.ds(a, 1), :] = lax.dot_general(
            pn, n3_ref[a],
            (((1,), (1,)), ((), ())),
            preferred_element_type=jnp.float32)


def _scores_call(p2, nodes3):
    return pl.pallas_call(
        _scores_body,
        out_shape=jax.ShapeDtypeStruct((SROW, SCOL), jnp.float32),
    )(p2, nodes3)


def _gru_body(sc_ref, nodes_ref, w_ref,
              wz_ref, uz_ref, bz_ref, wr_ref, ur_ref, br_ref,
              wh_ref, uh_ref, bh_ref, out_ref, x_ref):
    lin = (lax.broadcasted_iota(jnp.int32, (SROW, SCOL), 0) * SCOL
           + lax.broadcasted_iota(jnp.int32, (SROW, SCOL), 1))

    def step(t, S):
        m = jnp.max(S)
        amin = jnp.min(jnp.where(S == m, lin, jnp.int32(2**30)))
        row = nodes_ref[pl.ds(amin, 1), :]
        x_ref[pl.ds(t, 1), :] = row * jnp.tanh(m)
        return jnp.where(lin == amin, -jnp.inf, S)

    lax.fori_loop(0, K, step, sc_ref[...])

    X = x_ref[...]
    H = w_ref[...]
    dgT = lambda A, B: lax.dot_general(
        A, B, (((1,), (1,)), ((), ())), preferred_element_type=jnp.float32)
    mm = lambda A, B: jnp.dot(A, B, preferred_element_type=jnp.float32)
    Z = jax.nn.sigmoid(dgT(wz_ref[...], X) + mm(uz_ref[...], H) + bz_ref[...])
    R = jax.nn.sigmoid(dgT(wr_ref[...], X) + mm(ur_ref[...], H) + br_ref[...])
    Ht = jnp.tanh(dgT(wh_ref[...], X) + mm(uh_ref[...], R * H) + bh_ref[...])
    out_ref[...] = (1.0 - Z) * H + Z * Ht


def _gru_call(scores, nodes, weights, Wz, Uz, bz, Wr, Ur, br, Wh, Uh, bh):
    return pl.pallas_call(
        _gru_body,
        out_shape=jax.ShapeDtypeStruct((D, K), jnp.float32),
        scratch_shapes=[pltpu.VMEM((K, D), jnp.float32)],
    )(scores, nodes, weights, Wz, Uz, bz, Wr, Ur, br, Wh, Uh, bh)


_MM_BLK = 1000


def _mm_body(a0_ref, a1_ref, w_ref, o_ref):
    acc = a0_ref[...] + a1_ref[...]
    o_ref[...] = jnp.maximum(
        jnp.dot(acc, w_ref[...], preferred_element_type=jnp.float32), 0.0)


def _mm_call(a0, a1, w):
    return pl.pallas_call(
        _mm_body,
        grid=(N // _MM_BLK,),
        in_specs=[
            pl.BlockSpec((_MM_BLK, D), lambda i: (i, 0)),
            pl.BlockSpec((_MM_BLK, D), lambda i: (i, 0)),
            pl.BlockSpec((D, K), lambda i: (0, 0)),
        ],
        out_specs=pl.BlockSpec((_MM_BLK, K), lambda i: (i, 0)),
        out_shape=jax.ShapeDtypeStruct((N, K), jnp.float32),
    )(a0, a1, w)


def kernel(nodes, adj_values, weights, p, Wz, Uz, bz, Wr, Ur, br, Wh, Uh, bh,
           edge_index):
    spread = (jnp.arange(EPAD, dtype=jnp.int32) * 97) % N
    src3 = jnp.concatenate([edge_index[0], spread]).reshape(NW, 2, PH, CH)
    dst3 = jnp.concatenate([edge_index[1], spread]).reshape(NW, 2, PH, CH)
    adj3 = jnp.concatenate(
        [adj_values, jnp.zeros((EPAD,), jnp.float32)]).reshape(NW, 2, PH, CH)
    agg2 = _sc_agg(nodes, src3, dst3, adj3)
    scores = _scores_call(p.reshape(1, D), nodes.reshape(SROW, SCOL, D))
    weights_new = _gru_call(scores, nodes, weights,
                            Wz, Uz, bz, Wr, Ur, br, Wh, Uh, bh)
    nodes_new = _mm_call(agg2[:N], agg2[N:], weights_new)
    return (nodes_new, weights_new)


# speedup vs baseline: 9.4395x; 1.0711x over previous
import functools

import jax
import jax.numpy as jnp
from jax import lax
from jax.experimental import pallas as pl
from jax.experimental.pallas import tpu as pltpu
from jax.experimental.pallas import tpu_sc as plsc

N = 10000
D = 128
E = 320000
K = 128

NC = 2
NS = 16
NW = NC * NS
EPW = E // NW
CH = 64
PCH = 32
NPHASE = 5
NCHUNK = NPHASE * PCH
EPAD = NW * NCHUNK * CH - E
NBUF = 4
OWN = 624
TAIL = N - NS * OWN

SROW = 80
SCOL = 125


def _sc_agg(nodes, src3, dst3, adj3):
    mesh = plsc.VectorSubcoreMesh(core_axis_name="c", subcore_axis_name="s")

    @functools.partial(
        pl.kernel,
        out_type=jax.ShapeDtypeStruct((NC * N, D), jnp.float32),
        mesh=mesh,
        scratch_types=dict(
            agg_sh=pltpu.VMEM_SHARED((N, D), jnp.float32),
            ebufs=[pltpu.VMEM((CH, D), jnp.float32) for _ in range(NBUF)],
            sidx2=pltpu.VMEM((PCH, CH), jnp.int32),
            didx2=pltpu.VMEM((PCH, CH), jnp.int32),
            adj2=pltpu.VMEM((PCH, CH), jnp.float32),
            gsems=[pltpu.SemaphoreType.DMA for _ in range(NBUF)],
            ssems=[pltpu.SemaphoreType.DMA for _ in range(NBUF)],
        ),
    )
    def body(nodes_hbm, src_hbm, dst_hbm, adj_hbm, out_hbm,
             agg_sh, ebufs, sidx2, didx2, adj2, gsems, ssems):
        ebufa = ebufs[0]
        ebufb = ebufs[1]
        c = lax.axis_index("c")
        s = lax.axis_index("s")
        wid = s * NC + c

        zeros16 = jnp.zeros((16,), jnp.float32)

        def zrow(r, carry):
            for q in range(D // 16):
                ebufa[r, pl.ds(q * 16, 16)] = zeros16
            return carry

        lax.fori_loop(0, CH, zrow, 0)
        for z in range(OWN // CH):
            pltpu.sync_copy(ebufa, agg_sh.at[pl.ds(s * OWN + z * CH, CH)])
        pltpu.sync_copy(ebufa.at[pl.ds(0, OWN % CH)],
                        agg_sh.at[pl.ds(s * OWN + (OWN // CH) * CH, OWN % CH)])

        @pl.when(s == NS - 1)
        def _zero_tail():
            pltpu.sync_copy(ebufa.at[pl.ds(0, TAIL)],
                            agg_sh.at[pl.ds(NS * OWN, TAIL)])

        plsc.subcore_barrier()

        def gather_start(j, b):
            pltpu.make_async_copy(nodes_hbm.at[sidx2.at[j]],
                                  ebufs[b], gsems[b]).start()

        def gather_wait(j, b):
            pltpu.make_async_copy(nodes_hbm.at[sidx2.at[j]],
                                  ebufs[b], gsems[b]).wait()

        def scatter_start(j, b):
            pltpu.async_copy(ebufs[b], agg_sh.at[didx2.at[j]],
                             ssems[b], add=True)

        def scatter_wait(b):
            pltpu.make_async_copy(ebufs[b], agg_sh.at[didx2.at[0]],
                                  ssems[b]).wait()

        def scale(j, b):
            ebuf = ebufs[b]

            def grp(jj, carry):
                av = adj2[j, pl.ds(jj * 16, 16)]
                for l in range(16):
                    a = av.at[jnp.full((16,), l, jnp.int32)].get(
                        mode="promise_in_bounds")
                    r = jj * 16 + l
                    for q in range(D // 16):
                        ebuf[r, pl.ds(q * 16, 16)] = (
                            ebuf[r, pl.ds(q * 16, 16)] * a)
                return carry

            lax.fori_loop(0, CH // 16, grp, 0)

        def phase(ph, carry):
            pltpu.sync_copy(src_hbm.at[wid, ph], sidx2)
            pltpu.sync_copy(dst_hbm.at[wid, ph], didx2)
            pltpu.sync_copy(adj_hbm.at[wid, ph], adj2)
            gather_start(0, 0)
            gather_start(1, 1)

            def superblock(i, carry2):
                g0 = NBUF * i
                for b in range(NBUF):
                    g = g0 + b
                    gather_wait(g, b)
                    scale(g, b)
                    scatter_start(g, b)
                    bq = (b + 2) % NBUF

                    @pl.when(g + 2 < PCH)
                    def _prefetch():
                        @pl.when(g >= 2)
                        def _reuse_wait():
                            scatter_wait(bq)

                        gather_start(g + 2, bq)

                return carry2

            lax.fori_loop(0, PCH // NBUF, superblock, 0)
            for b in range(NBUF):
                scatter_wait(b)
            return carry

        lax.fori_loop(0, NPHASE, phase, 0)
        plsc.subcore_barrier()

        for z in range(OWN // CH):
            r0 = s * OWN + z * CH
            pltpu.sync_copy(agg_sh.at[pl.ds(r0, CH)], ebufa)
            pltpu.sync_copy(ebufa, out_hbm.at[pl.ds(c * N + r0, CH)])
        r0 = s * OWN + (OWN // CH) * CH
        pltpu.sync_copy(agg_sh.at[pl.ds(r0, OWN % CH)],
                        ebufa.at[pl.ds(0, OWN % CH)])
        pltpu.sync_copy(ebufa.at[pl.ds(0, OWN % CH)],
                        out_hbm.at[pl.ds(c * N + r0, OWN % CH)])

        @pl.when(s == NS - 1)
        def _write_tail():
            pltpu.sync_copy(agg_sh.at[pl.ds(NS * OWN, TAIL)],
                            ebufb.at[pl.ds(0, TAIL)])
            pltpu.sync_copy(ebufb.at[pl.ds(0, TAIL)],
                            out_hbm.at[pl.ds(c * N + NS * OWN, TAIL)])

    return body(nodes, src3, dst3, adj3)


def _scores_body(p_ref, n3_ref, o_ref):
    p = p_ref[...]
    pn = p / (jnp.sqrt(jnp.sum(p * p)) + 1e-8)
    for a in range(SROW):
        o_ref[pl.ds(a, 1), :] = lax.dot_general(
            pn, n3_ref[a],
            (((1,), (1,)), ((), ())),
            preferred_element_type=jnp.float32)


def _scores_call(p2, nodes3):
    return pl.pallas_call(
        _scores_body,
        out_shape=jax.ShapeDtypeStruct((SROW, SCOL), jnp.float32),
    )(p2, nodes3)


def _gru_body(sc_ref, nodes_ref, w_ref,
              wz_ref, uz_ref, bz_ref, wr_ref, ur_ref, br_ref,
              wh_ref, uh_ref, bh_ref, out_ref, x_ref):
    lin = (lax.broadcasted_iota(jnp.int32, (SROW, SCOL), 0) * SCOL
           + lax.broadcasted_iota(jnp.int32, (SROW, SCOL), 1))

    def step(t, S):
        m = jnp.max(S)
        amin = jnp.min(jnp.where(S == m, lin, jnp.int32(2**30)))
        row = nodes_ref[pl.ds(amin, 1), :]
        x_ref[pl.ds(t, 1), :] = row * jnp.tanh(m)
        return jnp.where(lin == amin, -jnp.inf, S)

    lax.fori_loop(0, K, step, sc_ref[...])

    X = x_ref[...]
    H = w_ref[...]
    dgT = lambda A, B: lax.dot_general(
        A, B, (((1,), (1,)), ((), ())), preferred_element_type=jnp.float32)
    mm = lambda A, B: jnp.dot(A, B, preferred_element_type=jnp.float32)
    Z = jax.nn.sigmoid(dgT(wz_ref[...], X) + mm(uz_ref[...], H) + bz_ref[...])
    R = jax.nn.sigmoid(dgT(wr_ref[...], X) + mm(ur_ref[...], H) + br_ref[...])
    Ht = jnp.tanh(dgT(wh_ref[...], X) + mm(uh_ref[...], R * H) + bh_ref[...])
    out_ref[...] = (1.0 - Z) * H + Z * Ht


def _gru_call(scores, nodes, weights, Wz, Uz, bz, Wr, Ur, br, Wh, Uh, bh):
    return pl.pallas_call(
        _gru_body,
        out_shape=jax.ShapeDtypeStruct((D, K), jnp.float32),
        scratch_shapes=[pltpu.VMEM((K, D), jnp.float32)],
    )(scores, nodes, weights, Wz, Uz, bz, Wr, Ur, br, Wh, Uh, bh)


_MM_BLK = 1000


def _mm_body(a0_ref, a1_ref, w_ref, o_ref):
    acc = a0_ref[...] + a1_ref[...]
    o_ref[...] = jnp.maximum(
        jnp.dot(acc, w_ref[...], preferred_element_type=jnp.float32), 0.0)


def _mm_call(a0, a1, w):
    return pl.pallas_call(
        _mm_body,
        grid=(N // _MM_BLK,),
        in_specs=[
            pl.BlockSpec((_MM_BLK, D), lambda i: (i, 0)),
            pl.BlockSpec((_MM_BLK, D), lambda i: (i, 0)),
            pl.BlockSpec((D, K), lambda i: (0, 0)),
        ],
        out_specs=pl.BlockSpec((_MM_BLK, K), lambda i: (i, 0)),
        out_shape=jax.ShapeDtypeStruct((N, K), jnp.float32),
    )(a0, a1, w)


def kernel(nodes, adj_values, weights, p, Wz, Uz, bz, Wr, Ur, br, Wh, Uh, bh,
           edge_index):
    spread = (jnp.arange(EPAD, dtype=jnp.int32) * 97) % N
    shape4 = (NW, NPHASE, PCH, CH)
    src3 = jnp.concatenate([edge_index[0], spread]).reshape(shape4)
    dst3 = jnp.concatenate([edge_index[1], spread]).reshape(shape4)
    adj3 = jnp.concatenate(
        [adj_values, jnp.zeros((EPAD,), jnp.float32)]).reshape(shape4)
    agg2 = _sc_agg(nodes, src3, dst3, adj3)
    scores = _scores_call(p.reshape(1, D), nodes.reshape(SROW, SCOL, D))
    weights_new = _gru_call(scores, nodes, weights,
                            Wz, Uz, bz, Wr, Ur, br, Wh, Uh, bh)
    nodes_new = _mm_call(agg2[:N], agg2[N:], weights_new)
    return (nodes_new, weights_new)


# speedup vs baseline: 10.1311x; 1.0733x over previous
import functools

import jax
import jax.numpy as jnp
from jax import lax
from jax.experimental import pallas as pl
from jax.experimental.pallas import tpu as pltpu
from jax.experimental.pallas import tpu_sc as plsc

N = 10000
D = 128
E = 320000
K = 128

NC = 2
NS = 16
NW = NC * NS
EPW = E // NW
CH = 64
PCH = 32
NPHASE = 5
NCHUNK = NPHASE * PCH
EPAD = NW * NCHUNK * CH - E
NBUF = 4
OWN = 624
TAIL = N - NS * OWN

SROW = 80
SCOL = 125


def _sc_agg(nodes, src3, dst3, adj3):
    mesh = plsc.VectorSubcoreMesh(core_axis_name="c", subcore_axis_name="s")

    @functools.partial(
        pl.kernel,
        out_type=jax.ShapeDtypeStruct((NC * N, D), jnp.float32),
        mesh=mesh,
        scratch_types=dict(
            agg_sh=pltpu.VMEM_SHARED((N, D), jnp.float32),
            ebufs=[pltpu.VMEM((CH, D), jnp.float32) for _ in range(NBUF)],
            sidx2=pltpu.VMEM((PCH, CH), jnp.int32),
            didx2=pltpu.VMEM((PCH, CH), jnp.int32),
            adj2=pltpu.VMEM((PCH, CH), jnp.float32),
            gsems=[pltpu.SemaphoreType.DMA for _ in range(NBUF)],
            ssems=[pltpu.SemaphoreType.DMA for _ in range(NBUF)],
        ),
    )
    def body(nodes_hbm, src_hbm, dst_hbm, adj_hbm, out_hbm,
             agg_sh, ebufs, sidx2, didx2, adj2, gsems, ssems):
        ebufa = ebufs[0]
        ebufb = ebufs[1]
        c = lax.axis_index("c")
        s = lax.axis_index("s")
        wid = s * NC + c

        zeros16 = jnp.zeros((16,), jnp.float32)

        def zrow(r, carry):
            for q in range(D // 16):
                ebufa[r, pl.ds(q * 16, 16)] = zeros16
            return carry

        lax.fori_loop(0, CH, zrow, 0)
        for z in range(OWN // CH):
            pltpu.sync_copy(ebufa, agg_sh.at[pl.ds(s * OWN + z * CH, CH)])
        pltpu.sync_copy(ebufa.at[pl.ds(0, OWN % CH)],
                        agg_sh.at[pl.ds(s * OWN + (OWN // CH) * CH, OWN % CH)])

        @pl.when(s == NS - 1)
        def _zero_tail():
            pltpu.sync_copy(ebufa.at[pl.ds(0, TAIL)],
                            agg_sh.at[pl.ds(NS * OWN, TAIL)])

        plsc.subcore_barrier()

        def gather_start(j, b):
            pltpu.make_async_copy(nodes_hbm.at[sidx2.at[j]],
                                  ebufs[b], gsems[b]).start()

        def gather_wait(j, b):
            pltpu.make_async_copy(nodes_hbm.at[sidx2.at[j]],
                                  ebufs[b], gsems[b]).wait()

        def scatter_start(j, b):
            pltpu.async_copy(ebufs[b], agg_sh.at[didx2.at[j]],
                             ssems[b], add=True)

        def scatter_wait(b):
            pltpu.make_async_copy(ebufs[b], agg_sh.at[didx2.at[0]],
                                  ssems[b]).wait()

        def scale(j, b):
            ebuf = ebufs[b]

            def grp(jj, carry):
                av = adj2[j, pl.ds(jj * 16, 16)]
                for l in range(16):
                    a = av.at[jnp.full((16,), l, jnp.int32)].get(
                        mode="promise_in_bounds")
                    r = jj * 16 + l
                    for q in range(D // 16):
                        ebuf[r, pl.ds(q * 16, 16)] = (
                            ebuf[r, pl.ds(q * 16, 16)] * a)
                return carry

            lax.fori_loop(0, CH // 16, grp, 0)

        def phase(ph, carry):
            pltpu.sync_copy(src_hbm.at[wid, ph], sidx2)
            pltpu.sync_copy(dst_hbm.at[wid, ph], didx2)
            pltpu.sync_copy(adj_hbm.at[wid, ph], adj2)
            gather_start(0, 0)
            gather_start(1, 1)
            gather_start(2, 2)

            def superblock(i, carry2):
                g0 = NBUF * i
                for b in range(NBUF):
                    g = g0 + b
                    gather_wait(g, b)
                    scale(g, b)
                    scatter_start(g, b)
                    bq = (b + 3) % NBUF

                    @pl.when(g + 3 < PCH)
                    def _prefetch():
                        @pl.when(g >= 1)
                        def _reuse_wait():
                            scatter_wait(bq)

                        gather_start(g + 3, bq)

                return carry2

            lax.fori_loop(0, PCH // NBUF, superblock, 0)
            for b in range(NBUF):
                scatter_wait(b)
            return carry

        lax.fori_loop(0, NPHASE, phase, 0)
        plsc.subcore_barrier()

        for z in range(OWN // CH):
            r0 = s * OWN + z * CH
            pltpu.sync_copy(agg_sh.at[pl.ds(r0, CH)], ebufa)
            pltpu.sync_copy(ebufa, out_hbm.at[pl.ds(c * N + r0, CH)])
        r0 = s * OWN + (OWN // CH) * CH
        pltpu.sync_copy(agg_sh.at[pl.ds(r0, OWN % CH)],
                        ebufa.at[pl.ds(0, OWN % CH)])
        pltpu.sync_copy(ebufa.at[pl.ds(0, OWN % CH)],
                        out_hbm.at[pl.ds(c * N + r0, OWN % CH)])

        @pl.when(s == NS - 1)
        def _write_tail():
            pltpu.sync_copy(agg_sh.at[pl.ds(NS * OWN, TAIL)],
                            ebufb.at[pl.ds(0, TAIL)])
            pltpu.sync_copy(ebufb.at[pl.ds(0, TAIL)],
                            out_hbm.at[pl.ds(c * N + NS * OWN, TAIL)])

    return body(nodes, src3, dst3, adj3)


def _scores_body(p_ref, n3_ref, o_ref):
    p = p_ref[...]
    pn = p / (jnp.sqrt(jnp.sum(p * p)) + 1e-8)
    for a in range(SROW):
        o_ref[pl.ds(a, 1), :] = lax.dot_general(
            pn, n3_ref[a],
            (((1,), (1,)), ((), ())),
            preferred_element_type=jnp.float32)


def _scores_call(p2, nodes3):
    return pl.pallas_call(
        _scores_body,
        out_shape=jax.ShapeDtypeStruct((SROW, SCOL), jnp.float32),
    )(p2, nodes3)


def _gru_body(sc_ref, nodes_ref, w_ref,
              wz_ref, uz_ref, bz_ref, wr_ref, ur_ref, br_ref,
              wh_ref, uh_ref, bh_ref, out_ref, x_ref):
    lin = (lax.broadcasted_iota(jnp.int32, (SROW, SCOL), 0) * SCOL
           + lax.broadcasted_iota(jnp.int32, (SROW, SCOL), 1))

    def step(t, S):
        m = jnp.max(S)
        amin = jnp.min(jnp.where(S == m, lin, jnp.int32(2**30)))
        row = nodes_ref[pl.ds(amin, 1), :]
        x_ref[pl.ds(t, 1), :] = row * jnp.tanh(m)
        return jnp.where(lin == amin, -jnp.inf, S)

    lax.fori_loop(0, K, step, sc_ref[...])

    X = x_ref[...]
    H = w_ref[...]
    dgT = lambda A, B: lax.dot_general(
        A, B, (((1,), (1,)), ((), ())), preferred_element_type=jnp.float32)
    mm = lambda A, B: jnp.dot(A, B, preferred_element_type=jnp.float32)
    Z = jax.nn.sigmoid(dgT(wz_ref[...], X) + mm(uz_ref[...], H) + bz_ref[...])
    R = jax.nn.sigmoid(dgT(wr_ref[...], X) + mm(ur_ref[...], H) + br_ref[...])
    Ht = jnp.tanh(dgT(wh_ref[...], X) + mm(uh_ref[...], R * H) + bh_ref[...])
    out_ref[...] = (1.0 - Z) * H + Z * Ht


def _gru_call(scores, nodes, weights, Wz, Uz, bz, Wr, Ur, br, Wh, Uh, bh):
    return pl.pallas_call(
        _gru_body,
        out_shape=jax.ShapeDtypeStruct((D, K), jnp.float32),
        scratch_shapes=[pltpu.VMEM((K, D), jnp.float32)],
    )(scores, nodes, weights, Wz, Uz, bz, Wr, Ur, br, Wh, Uh, bh)


_MM_BLK = 1000


def _mm_body(a0_ref, a1_ref, w_ref, o_ref):
    acc = a0_ref[...] + a1_ref[...]
    o_ref[...] = jnp.maximum(
        jnp.dot(acc, w_ref[...], preferred_element_type=jnp.float32), 0.0)


def _mm_call(a0, a1, w):
    return pl.pallas_call(
        _mm_body,
        grid=(N // _MM_BLK,),
        in_specs=[
            pl.BlockSpec((_MM_BLK, D), lambda i: (i, 0)),
            pl.BlockSpec((_MM_BLK, D), lambda i: (i, 0)),
            pl.BlockSpec((D, K), lambda i: (0, 0)),
        ],
        out_specs=pl.BlockSpec((_MM_BLK, K), lambda i: (i, 0)),
        out_shape=jax.ShapeDtypeStruct((N, K), jnp.float32),
    )(a0, a1, w)


def kernel(nodes, adj_values, weights, p, Wz, Uz, bz, Wr, Ur, br, Wh, Uh, bh,
           edge_index):
    spread = (jnp.arange(EPAD, dtype=jnp.int32) * 97) % N
    shape4 = (NW, NPHASE, PCH, CH)
    src3 = jnp.concatenate([edge_index[0], spread]).reshape(shape4)
    dst3 = jnp.concatenate([edge_index[1], spread]).reshape(shape4)
    adj3 = jnp.concatenate(
        [adj_values, jnp.zeros((EPAD,), jnp.float32)]).reshape(shape4)
    agg2 = _sc_agg(nodes, src3, dst3, adj3)
    scores = _scores_call(p.reshape(1, D), nodes.reshape(SROW, SCOL, D))
    weights_new = _gru_call(scores, nodes, weights,
                            Wz, Uz, bz, Wr, Ur, br, Wh, Uh, bh)
    nodes_new = _mm_call(agg2[:N], agg2[N:], weights_new)
    return (nodes_new, weights_new)
